# Initial kernel scaffold; baseline (speedup 1.0000x reference)
import jax
import jax.numpy as jnp
from jax.experimental import pallas as pl


def kernel(x, attention_rollout, Wq, bq, Wkv, bkv, Wp, bp):
    raise NotImplementedError("write your pallas kernel here")


# speedup vs baseline: 1.8338x; 1.8338x over previous
import functools

import jax
import jax.numpy as jnp
from jax import lax
from jax.experimental import pallas as pl
from jax.experimental.pallas import tpu as pltpu
from jax.experimental.pallas import tpu_sc as plsc

B, N, C, H = 2, 4096, 768, 12
DH = C // H
NSEL = max(1, int(0.1 * (N - 1)))
LPAD = 416
NCHUNK = (B * LPAD) // 16
NW = 32
SCALE = DH ** -0.5

def _sc_mesh():
  return plsc.VectorSubcoreMesh(core_axis_name="c", subcore_axis_name="s",
                                num_cores=2, num_subcores=16)


@functools.cache
def _sc_gather():
  @functools.partial(
      pl.kernel,
      mesh=_sc_mesh(),
      out_type=jax.ShapeDtypeStruct((B * LPAD, C), jnp.float32),
      scratch_types=[
          pltpu.VMEM((16,), jnp.int32),
          pltpu.VMEM((16, C), jnp.float32),
          pltpu.SemaphoreType.DMA,
      ],
  )
  def gather(x_hbm, idx_hbm, out_hbm, idx_v, rows_v, sem):
    wid = lax.axis_index("s") * 2 + lax.axis_index("c")

    def do(chunk):
      pltpu.sync_copy(idx_hbm.at[chunk], idx_v)
      pltpu.async_copy(x_hbm.at[idx_v], rows_v, sem).wait()
      pltpu.sync_copy(rows_v, out_hbm.at[pl.ds(chunk * 16, 16)])

    do(wid)

    @pl.when(wid + NW < NCHUNK)
    def _():
      do(wid + NW)

  return gather


@functools.cache
def _sc_scatter():
  @functools.partial(
      pl.kernel,
      mesh=_sc_mesh(),
      out_type=(),
      scratch_types=[
          pltpu.VMEM((16,), jnp.int32),
          pltpu.VMEM((16, C), jnp.float32),
          pltpu.SemaphoreType.DMA,
      ],
  )
  def scatter(out_ref, loc_hbm, idx_hbm, idx_v, rows_v, sem):
    wid = lax.axis_index("s") * 2 + lax.axis_index("c")

    def do(chunk):
      pltpu.sync_copy(idx_hbm.at[chunk], idx_v)
      pltpu.sync_copy(loc_hbm.at[pl.ds(chunk * 16, 16)], rows_v)
      pltpu.async_copy(rows_v, out_ref.at[idx_v], sem).wait()

    do(wid)

    @pl.when(wid + NW < NCHUNK)
    def _():
      do(wid + NW)

  return scatter


def _kv_body(x_ref, wkv_ref, bkv_ref, kv_ref, cp_ref):
  xb = x_ref[0].astype(jnp.bfloat16)
  kv = jnp.dot(xb, wkv_ref[...], preferred_element_type=jnp.float32)
  kv = kv + bkv_ref[...]
  kv_ref[0] = kv.astype(jnp.bfloat16)
  cp_ref[...] = x_ref[...]


_KV_NB = 8
_KV_BLK = N // _KV_NB


def _kv_call(x, wkv_t_bf, bkv2):
  return pl.pallas_call(
      _kv_body,
      grid=(B, _KV_NB),
      in_specs=[
          pl.BlockSpec((1, _KV_BLK, C), lambda b, nb: (b, nb, 0)),
          pl.BlockSpec((C, 2 * C), lambda b, nb: (0, 0)),
          pl.BlockSpec((1, 2 * C), lambda b, nb: (0, 0)),
      ],
      out_specs=[
          pl.BlockSpec((1, _KV_BLK, 2 * C), lambda b, nb: (b, nb, 0)),
          pl.BlockSpec((1, _KV_BLK, C), lambda b, nb: (b, nb, 0)),
      ],
      out_shape=[
          jax.ShapeDtypeStruct((B, N, 2 * C), jnp.bfloat16),
          jax.ShapeDtypeStruct((B, N, C), jnp.float32),
      ],
  )(x, wkv_t_bf, bkv2)


G = H // 2


def _attn_body(selq_ref, wq_ref, bq_ref, k_ref, v_ref, wp_ref, bp_ref,
               loc_ref, pacc_ref):
  g = pl.program_id(1)
  q2 = jnp.dot(selq_ref[0], wq_ref[...], preferred_element_type=jnp.float32)
  q2 = q2 + bq_ref[...]
  q2b = q2.astype(jnp.bfloat16)
  k2 = k_ref[0]
  v2 = v_ref[0]

  @pl.when(g == 0)
  def _():
    pacc_ref[...] = jnp.broadcast_to(bp_ref[...], (LPAD, C))

  for half in range(2):
    sl = slice(half * DH, (half + 1) * DH)
    logits = lax.dot_general(
        q2b[:, sl], k2[:, sl],
        dimension_numbers=(((1,), (1,)), ((), ())),
        preferred_element_type=jnp.float32)
    logits = logits * SCALE
    m = jnp.max(logits, axis=1, keepdims=True)
    p = jnp.exp(logits - m)
    s = jnp.sum(p, axis=1, keepdims=True)
    o = lax.dot_general(
        p.astype(jnp.bfloat16), v2[:, sl],
        dimension_numbers=(((1,), (0,)), ((), ())),
        preferred_element_type=jnp.float32)
    o = o / s
    pacc_ref[...] += jnp.dot(o.astype(jnp.bfloat16), wp_ref[sl, :],
                             preferred_element_type=jnp.float32)

  @pl.when(g == G - 1)
  def _():
    loc_ref[0] = pacc_ref[...]


def _attn_call(selq_bf, wq_t_bf, bq2, kv_bf, wp_t_bf, bp2):
  return pl.pallas_call(
      _attn_body,
      grid=(B, G),
      in_specs=[
          pl.BlockSpec((1, LPAD, C), lambda b, g: (b, 0, 0)),
          pl.BlockSpec((C, 2 * DH), lambda b, g: (0, g)),
          pl.BlockSpec((1, 2 * DH), lambda b, g: (0, g)),
          pl.BlockSpec((1, N, 2 * DH), lambda b, g: (b, 0, g)),
          pl.BlockSpec((1, N, 2 * DH), lambda b, g: (b, 0, g + G)),
          pl.BlockSpec((2 * DH, C), lambda b, g: (g, 0)),
          pl.BlockSpec((1, C), lambda b, g: (0, 0)),
      ],
      out_specs=pl.BlockSpec((1, LPAD, C), lambda b, g: (b, 0, 0)),
      out_shape=jax.ShapeDtypeStruct((B, LPAD, C), jnp.float32),
      scratch_shapes=[pltpu.VMEM((LPAD, C), jnp.float32)],
  )(selq_bf, wq_t_bf, bq2, kv_bf, kv_bf, wp_t_bf, bp2)


def kernel(x, attention_rollout, Wq, bq, Wkv, bkv, Wp, bp):
  cls_attention = attention_rollout[:, 0, 1:]
  _, top_idx = lax.top_k(cls_attention, NSEL)
  top_idx = top_idx + 1

  idxp = jnp.concatenate(
      [top_idx, jnp.broadcast_to(top_idx[:, NSEL - 1:NSEL], (B, LPAD - NSEL))],
      axis=1)
  flat_idx = (idxp + jnp.arange(B, dtype=idxp.dtype)[:, None] * N)
  flat_idx = flat_idx.reshape(NCHUNK, 16).astype(jnp.int32)

  x2 = x.reshape(B * N, C)
  selq = _sc_gather()(x2, flat_idx)
  selq_bf = selq.reshape(B, LPAD, C).astype(jnp.bfloat16)

  wkv_t_bf = Wkv.T.astype(jnp.bfloat16)
  wq_t_bf = Wq.T.astype(jnp.bfloat16)
  wp_t_bf = Wp.T.astype(jnp.bfloat16)

  kv_bf, out_cp = _kv_call(x, wkv_t_bf, bkv.reshape(1, 2 * C))
  loc = _attn_call(selq_bf, wq_t_bf, bq.reshape(1, C), kv_bf, wp_t_bf,
                   bp.reshape(1, C))

  out_ref = jax.new_ref(out_cp.reshape(B * N, C))
  _sc_scatter()(out_ref, loc.reshape(B * LPAD, C), flat_idx)
  return out_ref[...].reshape(B, N, C)
